# baseline (speedup 1.0000x reference)
import functools

import jax
import jax.numpy as jnp
from jax import lax
from jax.experimental import pallas as pl
from jax.experimental.pallas import tpu as pltpu
from jax.experimental.pallas import tpu_sc as plsc

_L = 16

_DNUMS = lax.GatherDimensionNumbers(
    offset_dims=(), collapsed_slice_dims=(0,), start_index_map=(0,))


def _lanesum(x):
    for k in range(4):
        perm = lax.iota(jnp.int32, _L) ^ (1 << k)
        x = x + lax.gather(x, perm[:, None], _DNUMS, slice_sizes=(1,),
                           mode=lax.GatherScatterMode.PROMISE_IN_BOUNDS)
    return x


def _fuse_body(r_ref, w_ref, out_ref):
    w = w_ref[...]
    denom = jnp.maximum(jnp.sqrt(jnp.sum(w * w, axis=1, keepdims=True)), 1e-12)
    out_ref[...] = jnp.concatenate([r_ref[...], w / denom], axis=1)


def _fuse_relations(R_w, W_w):
    n_rel, dim = R_w.shape
    return pl.pallas_call(
        _fuse_body,
        out_shape=jax.ShapeDtypeStruct((n_rel, 2 * dim), R_w.dtype),
    )(R_w, W_w)


def _transpose_body(a_ref, out_ref):
    dim = a_ref.shape[0]
    x = a_ref[...]
    r = lax.broadcasted_iota(jnp.int32, (dim, dim), 0)
    c = lax.broadcasted_iota(jnp.int32, (dim, dim), 1)
    eye = (r == c).astype(jnp.float32)
    out_ref[:, :dim] = lax.dot_general(
        x, eye, (((0,), (0,)), ((), ())),
        preferred_element_type=jnp.float32)


def _pack_entities(Et, blk):
    dim, n = Et.shape
    grid = (n + blk - 1) // blk
    return pl.pallas_call(
        _transpose_body,
        grid=(grid,),
        in_specs=[pl.BlockSpec((dim, blk), lambda i: (0, i))],
        out_specs=pl.BlockSpec((blk, 2 * dim), lambda i: (i, 0)),
        out_shape=jax.ShapeDtypeStruct((n, 2 * dim), jnp.float32),
    )(Et)


def _make_sc_kernel(batch, dim, n_entities, chunk):
    info = plsc.get_sparse_core_info()
    nc, ns = info.num_cores, info.num_subcores
    nw = nc * ns
    per_tile = batch // nw
    n_chunks = per_tile // chunk
    nvec = dim // _L
    mesh = plsc.VectorSubcoreMesh(core_axis_name="c", subcore_axis_name="s")

    @functools.partial(
        pl.kernel,
        out_type=jax.ShapeDtypeStruct((batch,), jnp.float32),
        mesh=mesh,
        scratch_types=[
            pltpu.VMEM_SHARED((ns, chunk), jnp.int32),
            pltpu.VMEM_SHARED((ns, chunk), jnp.int32),
            pltpu.SMEM((chunk,), jnp.int32),
            pltpu.SMEM((chunk,), jnp.int32),
            pltpu.VMEM((chunk,), jnp.int32),
            pltpu.VMEM((chunk, 2 * dim), jnp.float32),
            pltpu.VMEM((chunk, 2 * dim), jnp.float32),
            pltpu.VMEM((chunk, 2 * dim), jnp.float32),
            pltpu.VMEM((chunk,), jnp.float32),
            pltpu.SemaphoreType.DMA,
            pltpu.SemaphoreType.DMA,
        ],
    )
    def sc_kernel(h_hbm, t_hbm, r_hbm, rm_hbm, rw_hbm, out_hbm,
                  sh_h, sh_t, hs, ts, ridx, hrows, trows, rw2, outbuf,
                  sem, rsem):
        sid = lax.axis_index("s")
        wid = sid * nc + lax.axis_index("c")
        base = wid * per_tile
        lane = lax.iota(jnp.int32, _L)
        for c in range(n_chunks):
            off = base + c * chunk
            pltpu.sync_copy(r_hbm.at[pl.ds(off, chunk)], ridx)
            rw_cp = pltpu.async_copy(rw_hbm.at[ridx], rw2, sem)
            pltpu.sync_copy(h_hbm.at[pl.ds(off, chunk)], sh_h.at[sid])
            pltpu.sync_copy(t_hbm.at[pl.ds(off, chunk)], sh_t.at[sid])
            pltpu.sync_copy(sh_h.at[sid], hs)
            pltpu.sync_copy(sh_t.at[sid], ts)

            def fetch(i, _):
                pltpu.async_copy(rm_hbm.at[hs[i] // 2], hrows.at[i], rsem)
                pltpu.async_copy(rm_hbm.at[ts[i] // 2], trows.at[i], rsem)
                return 0

            lax.fori_loop(0, chunk, fetch, 0)
            pltpu.make_async_copy(rm_hbm.at[pl.ds(0, chunk)], hrows, rsem).wait()
            pltpu.make_async_copy(rm_hbm.at[pl.ds(0, chunk)], trows, rsem).wait()
            rw_cp.wait()

            def group(g, _):
                vec = None
                for j in range(_L):
                    i = g * _L + j
                    ho = (hs[i] % 2) * dim
                    to = (ts[i] % 2) * dim
                    d = [hrows[i, pl.ds(ho + k * _L, _L)]
                         - trows[i, pl.ds(to + k * _L, _L)]
                         for k in range(nvec)]
                    n = [rw2[i, pl.ds(dim + k * _L, _L)] for k in range(nvec)]
                    p = d[0] * n[0]
                    for k in range(1, nvec):
                        p = p + d[k] * n[k]
                    s = _lanesum(p)
                    acc = None
                    for k in range(nvec):
                        v = d[k] - s * n[k] + rw2[i, pl.ds(k * _L, _L)]
                        a = jnp.abs(v)
                        acc = a if acc is None else acc + a
                    tot = _lanesum(acc)
                    vec = tot if vec is None else jnp.where(lane == j, tot, vec)
                outbuf[pl.ds(g * _L, _L)] = vec
                return 0

            lax.fori_loop(0, chunk // _L, group, 0)
            pltpu.sync_copy(outbuf, out_hbm.at[pl.ds(off, chunk)])

    return sc_kernel


def kernel(h, r, t, E_w, R_w, W_w):
    batch = h.shape[0]
    n_entities, dim = E_w.shape
    RM = jnp.reshape(E_w, (n_entities // 2, 2 * dim))
    RW = _fuse_relations(R_w, W_w)
    sc = _make_sc_kernel(batch, dim, n_entities, chunk=128)
    return sc(h, t, r, RM, RW)


# speedup vs baseline: 1.6646x; 1.6646x over previous
import functools

import jax
import jax.numpy as jnp
from jax import lax
from jax.experimental import pallas as pl
from jax.experimental.pallas import tpu as pltpu
from jax.experimental.pallas import tpu_sc as plsc

_L = 16

_DNUMS = lax.GatherDimensionNumbers(
    offset_dims=(), collapsed_slice_dims=(0,), start_index_map=(0,))


def _lanesum(x):
    for k in range(4):
        perm = lax.iota(jnp.int32, _L) ^ (1 << k)
        x = x + lax.gather(x, perm[:, None], _DNUMS, slice_sizes=(1,),
                           mode=lax.GatherScatterMode.PROMISE_IN_BOUNDS)
    return x


def _fuse_body(r_ref, w_ref, out_ref):
    w = w_ref[...]
    denom = jnp.maximum(jnp.sqrt(jnp.sum(w * w, axis=1, keepdims=True)), 1e-12)
    out_ref[...] = jnp.concatenate([r_ref[...], w / denom], axis=1)


def _fuse_relations(R_w, W_w):
    n_rel, dim = R_w.shape
    return pl.pallas_call(
        _fuse_body,
        out_shape=jax.ShapeDtypeStruct((n_rel, 2 * dim), R_w.dtype),
    )(R_w, W_w)


def _make_sc_kernel(batch, dim, chunk):
    info = plsc.get_sparse_core_info()
    nc, ns = info.num_cores, info.num_subcores
    nw = nc * ns
    per_tile = batch // nw
    n_chunks = per_tile // chunk
    nvec = dim // _L
    mesh = plsc.VectorSubcoreMesh(core_axis_name="c", subcore_axis_name="s")

    @functools.partial(
        pl.kernel,
        out_type=jax.ShapeDtypeStruct((batch,), jnp.float32),
        mesh=mesh,
        scratch_types=[
            pltpu.VMEM_SHARED((ns, chunk), jnp.int32),
            pltpu.VMEM_SHARED((ns, chunk), jnp.int32),
            pltpu.SMEM((chunk,), jnp.int32),
            pltpu.SMEM((chunk,), jnp.int32),
            pltpu.VMEM((2, chunk), jnp.int32),
            pltpu.VMEM((2, chunk, dim), jnp.float32),
            pltpu.VMEM((2, chunk, dim), jnp.float32),
            pltpu.VMEM((2, chunk, 2 * dim), jnp.float32),
            pltpu.VMEM((chunk,), jnp.float32),
            pltpu.SemaphoreType.DMA,
            pltpu.SemaphoreType.DMA,
            pltpu.SemaphoreType.DMA,
            pltpu.SemaphoreType.DMA,
        ],
    )
    def sc_kernel(h_hbm, t_hbm, r_hbm, e_hbm, rw_hbm, out_hbm,
                  sh_h, sh_t, hs, ts, ridx, hrows, trows, rw2, outbuf,
                  sem0, sem1, rsem0, rsem1):
        sid = lax.axis_index("s")
        wid = sid * nc + lax.axis_index("c")
        base = wid * per_tile
        lane = lax.iota(jnp.int32, _L)
        sems = [sem0, sem1]
        rsems = [rsem0, rsem1]

        def stage_and_fire(c):
            b = c % 2
            off = base + c * chunk
            pltpu.sync_copy(r_hbm.at[pl.ds(off, chunk)], ridx.at[b])
            pltpu.async_copy(rw_hbm.at[ridx.at[b]], rw2.at[b], sems[b])
            pltpu.sync_copy(h_hbm.at[pl.ds(off, chunk)], sh_h.at[sid])
            pltpu.sync_copy(t_hbm.at[pl.ds(off, chunk)], sh_t.at[sid])
            pltpu.sync_copy(sh_h.at[sid], hs)
            pltpu.sync_copy(sh_t.at[sid], ts)

            def fetch(i, _):
                pltpu.async_copy(e_hbm.at[hs[i]], hrows.at[b, i], rsems[b])
                pltpu.async_copy(e_hbm.at[ts[i]], trows.at[b, i], rsems[b])
                return 0

            lax.fori_loop(0, chunk, fetch, 0)

        stage_and_fire(0)
        for c in range(n_chunks):
            b = c % 2
            if c + 1 < n_chunks:
                stage_and_fire(c + 1)
            pltpu.make_async_copy(
                e_hbm.at[pl.ds(0, chunk)], hrows.at[b], rsems[b]).wait()
            pltpu.make_async_copy(
                e_hbm.at[pl.ds(0, chunk)], trows.at[b], rsems[b]).wait()
            pltpu.make_async_copy(
                rw_hbm.at[pl.ds(0, chunk)], rw2.at[b], sems[b]).wait()

            def group(g, _, b=b):
                vec = None
                for j in range(_L):
                    i = g * _L + j
                    d = [hrows[b, i, pl.ds(k * _L, _L)]
                         - trows[b, i, pl.ds(k * _L, _L)]
                         for k in range(nvec)]
                    n = [rw2[b, i, pl.ds(dim + k * _L, _L)] for k in range(nvec)]
                    p = d[0] * n[0]
                    for k in range(1, nvec):
                        p = p + d[k] * n[k]
                    s = _lanesum(p)
                    acc = None
                    for k in range(nvec):
                        v = d[k] - s * n[k] + rw2[b, i, pl.ds(k * _L, _L)]
                        a = jnp.abs(v)
                        acc = a if acc is None else acc + a
                    tot = _lanesum(acc)
                    vec = tot if vec is None else jnp.where(lane == j, tot, vec)
                outbuf[pl.ds(g * _L, _L)] = vec
                return 0

            lax.fori_loop(0, chunk // _L, group, 0)
            pltpu.sync_copy(outbuf, out_hbm.at[pl.ds(base + c * chunk, chunk)])

    return sc_kernel


def kernel(h, r, t, E_w, R_w, W_w):
    batch = h.shape[0]
    dim = E_w.shape[1]
    RW = _fuse_relations(R_w, W_w)
    sc = _make_sc_kernel(batch, dim, chunk=128)
    return sc(h, t, r, E_w, RW)
